# baseline (speedup 1.0000x reference)
import functools

import jax
import jax.numpy as jnp
from jax import lax
from jax.experimental import pallas as pl
from jax.experimental.pallas import tpu as pltpu
from jax.experimental.pallas import tpu_sc as plsc

N = 50000
E = 800000
H = 64
WC = 8
NC = 2
NS = 16
LANES = 16
HN = N // NC
ACC_ROWS = HN + 8
GW = 128
E_PAD = 802816
NBLK = E_PAD // GW
NBT = NBLK // NS
GCH = 2
GBLK = GCH * GW
NGB = E_PAD // GBLK // (NC * NS)
CCH = 8
BE = 4096
EG = E_PAD // BE
BN = 2000

_f32 = jnp.float32


def _leaky(v):
    return jnp.maximum(v, 0.2 * v)


def _dot(a, b):
    return jnp.dot(a, b, preferred_element_type=_f32,
                   precision=jax.lax.Precision.HIGHEST)


def _sc_mesh():
    return plsc.VectorSubcoreMesh(core_axis_name="core",
                                  subcore_axis_name="subcore")


_SC_PARAMS = pltpu.CompilerParams(use_tc_tiling_on_sc=False)


def _transform_row(i_ref, row_in, l_ref, row_out, base):
    for t in range(GW // LANES):
        v = i_ref[pl.ds(row_in, 1), pl.ds(t * LANES, LANES)]
        local = v - base
        local = jnp.where((local < 0) | (local >= HN), HN, local)
        l_ref[pl.ds(row_out, 1), pl.ds(t * LANES, LANES)] = local


def _zero_acc(zeros_hbm, acc, sid, rows, zr):
    per_tile = rows // NS
    ncp = per_tile // zr + 1
    for c in range(ncp):
        start = jnp.minimum(sid * per_tile + c * zr, rows - zr)
        pltpu.sync_copy(zeros_hbm, acc.at[pl.ds(start, zr)])


def _acc_writeout(acc, out_hbm, cid, sid, width):
    del width
    for k in range(13):
        c = k * NS + sid

        @pl.when(c < 200)
        def _():
            pltpu.sync_copy(acc.at[pl.ds(c * 125, 125)],
                            out_hbm.at[cid, pl.ds(c * 125, 125)])


def _sc_scatter_sum(vals, idx2, zeros64):
    @functools.partial(
        pl.kernel,
        out_type=jax.ShapeDtypeStruct((NC, HN, H), _f32),
        mesh=_sc_mesh(),
        compiler_params=_SC_PARAMS,
        scratch_types=[
            pltpu.VMEM_SHARED((ACC_ROWS, H), _f32),
            pltpu.VMEM((3, GW, H), _f32),
            pltpu.VMEM((3, 1, GW), jnp.int32),
            pltpu.VMEM((3, 1, GW), jnp.int32),
        ] + [pltpu.SemaphoreType.DMA] * 9,
    )
    def kern(vals_hbm, idx_hbm, z_hbm, out_hbm, acc, vbuf, ibuf, lbuf,
             sv0, sv1, sv2, sx0, sx1, sx2, ss0, ss1, ss2):
        sv = (sv0, sv1, sv2)
        sx = (sx0, sx1, sx2)
        ss = (ss0, ss1, ss2)
        cid = lax.axis_index("core")
        sid = lax.axis_index("subcore")
        base = cid * HN
        tb = sid * NBT
        _zero_acc(z_hbm, acc, sid, ACC_ROWS, GW)
        plsc.subcore_barrier()

        def issue_loads(b, k):
            pltpu.async_copy(vals_hbm.at[pl.ds((tb + b) * GW, GW)],
                             vbuf.at[k], sv[k])
            pltpu.async_copy(idx_hbm.at[pl.ds(tb + b, 1)], ibuf.at[k], sx[k])

        def wait_loads(k):
            pltpu.make_async_copy(vals_hbm.at[pl.ds(0, GW)],
                                  vbuf.at[k], sv[k]).wait()
            pltpu.make_async_copy(idx_hbm.at[pl.ds(0, 1)],
                                  ibuf.at[k], sx[k]).wait()

        def wait_scatter(k):
            pltpu.make_async_copy(vbuf.at[k], acc.at[lbuf.at[k, 0]],
                                  ss[k]).wait()

        def half(b, k, dyn):
            kp = (k - 1) % 3
            wait_loads(k)
            _transform_row(ibuf.at[k], 0, lbuf.at[k], 0, base)
            pltpu.async_copy(vbuf.at[k], acc.at[lbuf.at[k, 0]], ss[k],
                             add=True)
            if dyn:
                @pl.when(b >= 1)
                def _():
                    wait_scatter(kp)

                @pl.when(b + 2 < NBT)
                def _():
                    issue_loads(b + 2, kp)
            else:
                wait_scatter(kp)

        issue_loads(0, 0)
        issue_loads(1, 1)

        @pl.loop(0, NBT - 2, step=3)
        def _(g):
            for j in range(3):
                half(g + j, j, True)

        half(NBT - 2, (NBT - 2) % 3, False)
        half(NBT - 1, (NBT - 1) % 3, False)
        wait_scatter((NBT - 1) % 3)
        plsc.subcore_barrier()
        _acc_writeout(acc, out_hbm, cid, sid, H)

    return kern(vals, idx2, zeros64).reshape(N, H)


def _sc_count2(idxd2, idxs2, ones8, zeros8):
    @functools.partial(
        pl.kernel,
        out_type=[jax.ShapeDtypeStruct((NC, HN, WC), _f32),
                  jax.ShapeDtypeStruct((NC, HN, WC), _f32)],
        mesh=_sc_mesh(),
        compiler_params=_SC_PARAMS,
        scratch_types=[
            pltpu.VMEM_SHARED((ACC_ROWS, WC), _f32),
            pltpu.VMEM_SHARED((ACC_ROWS, WC), _f32),
            pltpu.VMEM((GW, WC), _f32),
            pltpu.VMEM((CCH, GW), jnp.int32),
            pltpu.VMEM((CCH, GW), jnp.int32),
            pltpu.SemaphoreType.DMA,
        ],
    )
    def kern(id_hbm, is_hbm, ones_hbm, z_hbm, outd_hbm, outs_hbm,
             accd, accs, ones_v, lbd, lbs, sem):
        cid = lax.axis_index("core")
        sid = lax.axis_index("subcore")
        base = cid * HN
        pltpu.sync_copy(ones_hbm, ones_v)
        _zero_acc(z_hbm, accd, sid, ACC_ROWS, GW)
        _zero_acc(z_hbm, accs, sid, ACC_ROWS, GW)
        plsc.subcore_barrier()

        def body(id_vmem, is_vmem):
            for j in range(CCH):
                _transform_row(id_vmem, j, lbd, j, base)
                _transform_row(is_vmem, j, lbs, j, base)
            for j in range(CCH):
                pltpu.async_copy(ones_v, accd.at[lbd.at[j]], sem, add=True)
                pltpu.async_copy(ones_v, accs.at[lbs.at[j]], sem, add=True)
            for j in range(CCH):
                pltpu.make_async_copy(ones_v, accd.at[lbd.at[j]], sem).wait()
                pltpu.make_async_copy(ones_v, accs.at[lbs.at[j]], sem).wait()

        pltpu.emit_pipeline(
            body,
            grid=(NBLK // CCH,),
            in_specs=[pl.BlockSpec((CCH, GW), lambda i: (i, 0)),
                      pl.BlockSpec((CCH, GW), lambda i: (i, 0))],
            out_specs=[],
            core_axis_name="subcore",
            dimension_semantics=(pltpu.PARALLEL,),
        )(id_hbm, is_hbm)
        plsc.subcore_barrier()
        _acc_writeout(accd, outd_hbm, cid, sid, WC)
        _acc_writeout(accs, outs_hbm, cid, sid, WC)

    cd, cs = kern(idxd2, idxs2, ones8, zeros8)
    return cd.reshape(N, WC), cs.reshape(N, WC)


def _sc_gather2(ta, tb, src2, dst2):
    @functools.partial(
        pl.kernel,
        out_type=[jax.ShapeDtypeStruct((E_PAD, H), _f32),
                  jax.ShapeDtypeStruct((E_PAD, H), _f32)],
        mesh=_sc_mesh(),
        compiler_params=_SC_PARAMS,
        scratch_types=[
            pltpu.VMEM((2, GBLK, H), _f32),
            pltpu.VMEM((2, GBLK, H), _f32),
            pltpu.VMEM((2, GCH, GW), jnp.int32),
            pltpu.VMEM((2, GCH, GW), jnp.int32),
        ] + [pltpu.SemaphoreType.DMA] * 6,
    )
    def kern(ta_hbm, tb_hbm, si_hbm, di_hbm, ga_hbm, gb_hbm,
             gabuf, gbbuf, sibuf, dibuf, si0, si1, sg0, sg1, so0, so1):
        s_i = (si0, si1)
        s_g = (sg0, sg1)
        s_o = (so0, so1)
        cid = lax.axis_index("core")
        sid = lax.axis_index("subcore")
        wid = cid * NS + sid
        tbb = wid * NGB

        def issue_idx(b, k):
            pltpu.async_copy(si_hbm.at[pl.ds((tbb + b) * GCH, GCH)],
                             sibuf.at[k], s_i[k])
            pltpu.async_copy(di_hbm.at[pl.ds((tbb + b) * GCH, GCH)],
                             dibuf.at[k], s_i[k])

        def wait_idx(k):
            pltpu.make_async_copy(si_hbm.at[pl.ds(0, GCH)],
                                  sibuf.at[k], s_i[k]).wait()
            pltpu.make_async_copy(di_hbm.at[pl.ds(0, GCH)],
                                  dibuf.at[k], s_i[k]).wait()

        def issue_gathers(k):
            for c in range(GCH):
                pltpu.async_copy(ta_hbm.at[sibuf.at[k, c]],
                                 gabuf.at[k, pl.ds(c * GW, GW)], s_g[k])
                pltpu.async_copy(tb_hbm.at[dibuf.at[k, c]],
                                 gbbuf.at[k, pl.ds(c * GW, GW)], s_g[k])

        def wait_gathers(k):
            for c in range(GCH):
                pltpu.make_async_copy(ta_hbm.at[sibuf.at[k, c]],
                                      gabuf.at[k, pl.ds(c * GW, GW)],
                                      s_g[k]).wait()
                pltpu.make_async_copy(tb_hbm.at[dibuf.at[k, c]],
                                      gbbuf.at[k, pl.ds(c * GW, GW)],
                                      s_g[k]).wait()

        def issue_outs(b, k):
            pltpu.async_copy(gabuf.at[k],
                             ga_hbm.at[pl.ds((tbb + b) * GBLK, GBLK)], s_o[k])
            pltpu.async_copy(gbbuf.at[k],
                             gb_hbm.at[pl.ds((tbb + b) * GBLK, GBLK)], s_o[k])

        def wait_outs(k):
            pltpu.make_async_copy(gabuf.at[k],
                                  ga_hbm.at[pl.ds(0, GBLK)], s_o[k]).wait()
            pltpu.make_async_copy(gbbuf.at[k],
                                  gb_hbm.at[pl.ds(0, GBLK)], s_o[k]).wait()

        issue_idx(0, 0)

        @pl.loop(0, NGB, step=2)
        def _(g):
            for j in range(2):
                b = g + j
                k = j
                kp = 1 - j
                wait_idx(k)

                @pl.when(b >= 2)
                def _():
                    wait_outs(k)

                issue_gathers(k)

                @pl.when(b >= 1)
                def _():
                    wait_gathers(kp)
                    issue_outs(b - 1, kp)

                @pl.when(b + 1 < NGB)
                def _():
                    issue_idx(b + 1, kp)

        wait_gathers(1)
        issue_outs(NGB - 1, 1)
        wait_outs(0)
        wait_outs(1)

    return kern(ta, tb, src2, dst2)


def _w_spec(r, c):
    return pl.BlockSpec((r, c), lambda i: (0, 0))


def _tc_enc_node(x_pad, w_src, w_dst):
    def body(x_ref, ws_ref, wd_ref, xa_ref, xb_ref):
        xv = x_ref[...]
        xa_ref[...] = _dot(xv, ws_ref[...])
        xb_ref[...] = _dot(xv, wd_ref[...])

    return pl.pallas_call(
        body,
        grid=(N // BN,),
        in_specs=[pl.BlockSpec((BN, 16), lambda i: (i, 0)),
                  _w_spec(16, H), _w_spec(16, H)],
        out_specs=[pl.BlockSpec((BN, H), lambda i: (i, 0)),
                   pl.BlockSpec((BN, H), lambda i: (i, 0))],
        out_shape=[jax.ShapeDtypeStruct((N, H), _f32),
                   jax.ShapeDtypeStruct((N, H), _f32)],
    )(x_pad, w_src, w_dst)


def _tc_node(a, b, cnt_d, cnt_s, wi, wo):
    def body(a_ref, b_ref, cd_ref, cs_ref, wi_ref, wo_ref, ta_ref, tb_ref):
        cd = jnp.maximum(cd_ref[:, 0:1], 1.0)
        cs = jnp.maximum(cs_ref[:, 0:1], 1.0)
        ta_ref[...] = _dot(a_ref[...] / cd, wi_ref[...])
        tb_ref[...] = _dot(b_ref[...] / cs, wo_ref[...])

    return pl.pallas_call(
        body,
        grid=(N // BN,),
        in_specs=[pl.BlockSpec((BN, H), lambda i: (i, 0)),
                  pl.BlockSpec((BN, H), lambda i: (i, 0)),
                  pl.BlockSpec((BN, WC), lambda i: (i, 0)),
                  pl.BlockSpec((BN, WC), lambda i: (i, 0)),
                  _w_spec(H, H), _w_spec(H, H)],
        out_specs=[pl.BlockSpec((BN, H), lambda i: (i, 0)),
                   pl.BlockSpec((BN, H), lambda i: (i, 0))],
        out_shape=[jax.ShapeDtypeStruct((N, H), _f32),
                   jax.ShapeDtypeStruct((N, H), _f32)],
    )(a, b, cnt_d, cnt_s, wi, wo)


def _tc_combine_enc(e_pad, ga, gb, w_e_pad, enc_b, ws1, bs1):
    def body(e_ref, ga_ref, gb_ref, we_ref, eb_ref, w_ref, b_ref,
             h_ref, s_ref):
        hv = _leaky(_dot(e_ref[...], we_ref[...]) + eb_ref[...]
                    + ga_ref[...] + gb_ref[...])
        h_ref[...] = hv
        s_ref[...] = _dot(hv, w_ref[...]) + b_ref[...]

    return pl.pallas_call(
        body,
        grid=(EG,),
        in_specs=[pl.BlockSpec((BE, 8), lambda i: (i, 0)),
                  pl.BlockSpec((BE, H), lambda i: (i, 0)),
                  pl.BlockSpec((BE, H), lambda i: (i, 0)),
                  _w_spec(8, H), _w_spec(1, H), _w_spec(H, H), _w_spec(1, H)],
        out_specs=[pl.BlockSpec((BE, H), lambda i: (i, 0)),
                   pl.BlockSpec((BE, H), lambda i: (i, 0))],
        out_shape=[jax.ShapeDtypeStruct((E_PAD, H), _f32),
                   jax.ShapeDtypeStruct((E_PAD, H), _f32)],
    )(e_pad, ga, gb, w_e_pad, enc_b, ws1, bs1)


def _tc_combine_conv(s, ga, gb, w_next, b_next):
    def body(s_ref, ga_ref, gb_ref, w_ref, b_ref, h_ref, so_ref):
        hv = _leaky(s_ref[...] + ga_ref[...] + gb_ref[...])
        h_ref[...] = hv
        so_ref[...] = _dot(hv, w_ref[...]) + b_ref[...]

    return pl.pallas_call(
        body,
        grid=(EG,),
        in_specs=[pl.BlockSpec((BE, H), lambda i: (i, 0)),
                  pl.BlockSpec((BE, H), lambda i: (i, 0)),
                  pl.BlockSpec((BE, H), lambda i: (i, 0)),
                  _w_spec(H, H), _w_spec(1, H)],
        out_specs=[pl.BlockSpec((BE, H), lambda i: (i, 0)),
                   pl.BlockSpec((BE, H), lambda i: (i, 0))],
        out_shape=[jax.ShapeDtypeStruct((E_PAD, H), _f32),
                   jax.ShapeDtypeStruct((E_PAD, H), _f32)],
    )(s, ga, gb, w_next, b_next)


def _tc_combine_mlp(s, ga, gb, m1w, m1b, m2w, m2b, m3w, m3b, m4w, m4b):
    def body(s_ref, ga_ref, gb_ref, w1_ref, b1_ref, w2_ref, b2_ref,
             w3_ref, b3_ref, w4_ref, b4_ref, o_ref):
        hv = _leaky(s_ref[...] + ga_ref[...] + gb_ref[...])
        hv = _leaky(_dot(hv, w1_ref[...]) + b1_ref[...])
        hv = _leaky(_dot(hv, w2_ref[...]) + b2_ref[...])
        hv = _leaky(_dot(hv, w3_ref[...]) + b3_ref[...])
        o_ref[...] = jnp.sum(hv * w4_ref[...], axis=1) + b4_ref[0, 0]

    return pl.pallas_call(
        body,
        grid=(EG,),
        in_specs=[pl.BlockSpec((BE, H), lambda i: (i, 0)),
                  pl.BlockSpec((BE, H), lambda i: (i, 0)),
                  pl.BlockSpec((BE, H), lambda i: (i, 0)),
                  _w_spec(H, H), _w_spec(1, H),
                  _w_spec(H, H), _w_spec(1, H),
                  _w_spec(H, 32), _w_spec(1, 32),
                  _w_spec(1, 32),
                  pl.BlockSpec((1, 1), lambda i: (0, 0),
                               memory_space=pltpu.SMEM)],
        out_specs=pl.BlockSpec((BE,), lambda i: (i,)),
        out_shape=jax.ShapeDtypeStruct((E,), _f32),
    )(s, ga, gb, m1w, m1b, m2w, m2b, m3w, m3b, m4w, m4b)


def kernel(x, edge_index, raw_edge_attr, enc_W, enc_b,
           c1_Ws, c1_bs, c1_Wi, c1_Wo,
           c2_Ws, c2_bs, c2_Wi, c2_Wo,
           c3_Ws, c3_bs, c3_Wi, c3_Wo,
           m1_W, m1_b, m2_W, m2_b, m3_W, m3_b, m4_W, m4_b):
    src = edge_index[0].astype(jnp.int32)
    dst = edge_index[1].astype(jnp.int32)
    npad = E_PAD - E
    pad_g = jnp.zeros((npad,), jnp.int32)
    pad_s = jnp.full((npad,), N, jnp.int32)
    src_g = jnp.concatenate([src, pad_g]).reshape(NBLK, GW)
    dst_g = jnp.concatenate([dst, pad_g]).reshape(NBLK, GW)
    src_s = jnp.concatenate([src, pad_s]).reshape(NBLK, GW)
    dst_s = jnp.concatenate([dst, pad_s]).reshape(NBLK, GW)

    zeros64 = jnp.zeros((GW, H), _f32)
    zeros8 = jnp.zeros((GW, WC), _f32)
    ones8 = jnp.ones((GW, WC), _f32)

    x_pad = jnp.pad(x, ((0, 0), (0, 3)))
    w_src = jnp.pad(enc_W[0:13], ((0, 3), (0, 0)))
    w_dst = jnp.pad(enc_W[13:26], ((0, 3), (0, 0)))
    e_pad = jnp.pad(raw_edge_attr, ((0, 0), (0, 7)))
    w_e_pad = jnp.pad(enc_W[26:27], ((0, 7), (0, 0)))

    cnt_d, cnt_s = _sc_count2(dst_s, src_s, ones8, zeros8)

    xa, xb = _tc_enc_node(x_pad, w_src, w_dst)
    ga, gb = _sc_gather2(xa, xb, src_g, dst_g)
    h, s = _tc_combine_enc(e_pad, ga, gb, w_e_pad, enc_b.reshape(1, H),
                           c1_Ws, c1_bs.reshape(1, H))

    convs = [(c1_Wi, c1_Wo, c2_Ws, c2_bs), (c2_Wi, c2_Wo, c3_Ws, c3_bs)]
    for wi, wo, ws_n, bs_n in convs:
        a = _sc_scatter_sum(h, dst_s, zeros64)
        b = _sc_scatter_sum(h, src_s, zeros64)
        ta, tb = _tc_node(a, b, cnt_d, cnt_s, wi, wo)
        ga, gb = _sc_gather2(ta, tb, src_g, dst_g)
        h, s = _tc_combine_conv(s, ga, gb, ws_n, bs_n.reshape(1, H))

    a = _sc_scatter_sum(h, dst_s, zeros64)
    b = _sc_scatter_sum(h, src_s, zeros64)
    ta, tb = _tc_node(a, b, cnt_d, cnt_s, c3_Wi, c3_Wo)
    ga, gb = _sc_gather2(ta, tb, src_g, dst_g)
    out = _tc_combine_mlp(s, ga, gb,
                          m1_W, m1_b.reshape(1, H),
                          m2_W, m2_b.reshape(1, H),
                          m3_W, m3_b.reshape(1, 32),
                          m4_W.reshape(1, 32), m4_b.reshape(1, 1))
    return out


# speedup vs baseline: 4.9900x; 4.9900x over previous
import functools

import jax
import jax.numpy as jnp
from jax import lax
from jax.experimental import pallas as pl
from jax.experimental.pallas import tpu as pltpu
from jax.experimental.pallas import tpu_sc as plsc

N = 50000
E = 800000
H = 64
WC = 8
NC = 2
NS = 16
LANES = 16
HN = N // NC
ACC_ROWS = HN + 8
GW = 128
E_PAD = 802816
NBLK = E_PAD // GW
NBT = NBLK // NS
GCH = 2
GBLK = GCH * GW
NGB = E_PAD // GBLK // (NC * NS)
CCH = 8
BE = 4096
EG = E_PAD // BE
BN = 2000

_f32 = jnp.float32


def _leaky(v):
    return jnp.maximum(v, 0.2 * v)


def _dot(a, b):
    return jnp.dot(a, b, preferred_element_type=_f32,
                   precision=jax.lax.Precision.HIGHEST)


def _sc_mesh():
    return plsc.VectorSubcoreMesh(core_axis_name="core",
                                  subcore_axis_name="subcore")


_SC_PARAMS = pltpu.CompilerParams(use_tc_tiling_on_sc=False)


def _transform_row(i_ref, row_in, l_ref, row_out, base):
    for t in range(GW // LANES):
        v = i_ref[pl.ds(row_in, 1), pl.ds(t * LANES, LANES)]
        local = v - base
        local = jnp.where((local < 0) | (local >= HN), HN, local)
        l_ref[pl.ds(row_out, 1), pl.ds(t * LANES, LANES)] = local


def _zero_acc(zeros_hbm, acc, sid, rows, zr):
    per_tile = rows // NS
    ncp = per_tile // zr + 1
    for c in range(ncp):
        start = jnp.minimum(sid * per_tile + c * zr, rows - zr)
        pltpu.sync_copy(zeros_hbm, acc.at[pl.ds(start, zr)])


def _acc_writeout(acc, out_hbm, cid, sid, width):
    del width
    for k in range(13):
        c = k * NS + sid

        @pl.when(c < 200)
        def _():
            pltpu.sync_copy(acc.at[pl.ds(c * 125, 125)],
                            out_hbm.at[cid, pl.ds(c * 125, 125)])


def _sc_scatter_sum(vals, idx2, zeros64):
    @functools.partial(
        pl.kernel,
        out_type=jax.ShapeDtypeStruct((NC, HN, H), _f32),
        mesh=_sc_mesh(),
        compiler_params=_SC_PARAMS,
        scratch_types=[
            pltpu.VMEM_SHARED((ACC_ROWS, H), _f32),
            pltpu.VMEM((1, GW), jnp.int32),
        ],
    )
    def kern(vals_hbm, idx_hbm, z_hbm, out_hbm, acc, idxl):
        cid = lax.axis_index("core")
        sid = lax.axis_index("subcore")
        base = cid * HN
        _zero_acc(z_hbm, acc, sid, ACC_ROWS, GW)
        plsc.subcore_barrier()

        def body(v_vmem, i_vmem):
            _transform_row(i_vmem, 0, idxl, 0, base)
            pltpu.sync_copy(v_vmem, acc.at[idxl.at[0]], add=True)

        pltpu.emit_pipeline(
            body,
            grid=(NBLK,),
            in_specs=[
                pl.BlockSpec((GW, H), lambda i: (i, 0)),
                pl.BlockSpec((1, GW), lambda i: (i, 0)),
            ],
            out_specs=[],
            core_axis_name="subcore",
            dimension_semantics=(pltpu.PARALLEL,),
        )(vals_hbm, idx_hbm)
        plsc.subcore_barrier()
        _acc_writeout(acc, out_hbm, cid, sid, H)

    return kern(vals, idx2, zeros64).reshape(N, H)


def _sc_count2(idxd2, idxs2, ones8, zeros8):
    @functools.partial(
        pl.kernel,
        out_type=[jax.ShapeDtypeStruct((NC, HN, WC), _f32),
                  jax.ShapeDtypeStruct((NC, HN, WC), _f32)],
        mesh=_sc_mesh(),
        compiler_params=_SC_PARAMS,
        scratch_types=[
            pltpu.VMEM_SHARED((ACC_ROWS, WC), _f32),
            pltpu.VMEM_SHARED((ACC_ROWS, WC), _f32),
            pltpu.VMEM((GW, WC), _f32),
            pltpu.VMEM((CCH, GW), jnp.int32),
            pltpu.VMEM((CCH, GW), jnp.int32),
            pltpu.SemaphoreType.DMA,
        ],
    )
    def kern(id_hbm, is_hbm, ones_hbm, z_hbm, outd_hbm, outs_hbm,
             accd, accs, ones_v, lbd, lbs, sem):
        cid = lax.axis_index("core")
        sid = lax.axis_index("subcore")
        base = cid * HN
        pltpu.sync_copy(ones_hbm, ones_v)
        _zero_acc(z_hbm, accd, sid, ACC_ROWS, GW)
        _zero_acc(z_hbm, accs, sid, ACC_ROWS, GW)
        plsc.subcore_barrier()

        def body(id_vmem, is_vmem):
            for j in range(CCH):
                _transform_row(id_vmem, j, lbd, j, base)
                _transform_row(is_vmem, j, lbs, j, base)
            for j in range(CCH):
                pltpu.async_copy(ones_v, accd.at[lbd.at[j]], sem, add=True)
                pltpu.async_copy(ones_v, accs.at[lbs.at[j]], sem, add=True)
            for j in range(CCH):
                pltpu.make_async_copy(ones_v, accd.at[lbd.at[j]], sem).wait()
                pltpu.make_async_copy(ones_v, accs.at[lbs.at[j]], sem).wait()

        pltpu.emit_pipeline(
            body,
            grid=(NBLK // CCH,),
            in_specs=[pl.BlockSpec((CCH, GW), lambda i: (i, 0)),
                      pl.BlockSpec((CCH, GW), lambda i: (i, 0))],
            out_specs=[],
            core_axis_name="subcore",
            dimension_semantics=(pltpu.PARALLEL,),
        )(id_hbm, is_hbm)
        plsc.subcore_barrier()
        _acc_writeout(accd, outd_hbm, cid, sid, WC)
        _acc_writeout(accs, outs_hbm, cid, sid, WC)

    cd, cs = kern(idxd2, idxs2, ones8, zeros8)
    return cd.reshape(N, WC), cs.reshape(N, WC)


def _sc_gather2(ta, tb, src2, dst2):
    @functools.partial(
        pl.kernel,
        out_type=[jax.ShapeDtypeStruct((E_PAD, H), _f32),
                  jax.ShapeDtypeStruct((E_PAD, H), _f32)],
        mesh=_sc_mesh(),
        compiler_params=_SC_PARAMS,
        scratch_types=[pltpu.SemaphoreType.DMA],
    )
    def kern(ta_hbm, tb_hbm, si_hbm, di_hbm, ga_hbm, gb_hbm, sem):
        def body(si_vmem, di_vmem, ga_vmem, gb_vmem):
            for c in range(GCH):
                pltpu.async_copy(ta_hbm.at[si_vmem.at[c]],
                                 ga_vmem.at[pl.ds(c * GW, GW)], sem)
                pltpu.async_copy(tb_hbm.at[di_vmem.at[c]],
                                 gb_vmem.at[pl.ds(c * GW, GW)], sem)
            for c in range(GCH):
                pltpu.make_async_copy(ta_hbm.at[si_vmem.at[c]],
                                      ga_vmem.at[pl.ds(c * GW, GW)],
                                      sem).wait()
                pltpu.make_async_copy(tb_hbm.at[di_vmem.at[c]],
                                      gb_vmem.at[pl.ds(c * GW, GW)],
                                      sem).wait()

        pltpu.emit_pipeline(
            body,
            grid=(NBLK // GCH,),
            in_specs=[pl.BlockSpec((GCH, GW), lambda i: (i, 0)),
                      pl.BlockSpec((GCH, GW), lambda i: (i, 0))],
            out_specs=[pl.BlockSpec((GBLK, H), lambda i: (i, 0)),
                       pl.BlockSpec((GBLK, H), lambda i: (i, 0))],
            core_axis_name=("core", "subcore"),
            dimension_semantics=(pltpu.PARALLEL,),
        )(si_hbm, di_hbm, ga_hbm, gb_hbm)

    return kern(ta, tb, src2, dst2)


def _w_spec(r, c):
    return pl.BlockSpec((r, c), lambda i: (0, 0))


def _tc_enc_node(x_pad, w_src, w_dst):
    def body(x_ref, ws_ref, wd_ref, xa_ref, xb_ref):
        xv = x_ref[...]
        xa_ref[...] = _dot(xv, ws_ref[...])
        xb_ref[...] = _dot(xv, wd_ref[...])

    return pl.pallas_call(
        body,
        grid=(N // BN,),
        in_specs=[pl.BlockSpec((BN, 16), lambda i: (i, 0)),
                  _w_spec(16, H), _w_spec(16, H)],
        out_specs=[pl.BlockSpec((BN, H), lambda i: (i, 0)),
                   pl.BlockSpec((BN, H), lambda i: (i, 0))],
        out_shape=[jax.ShapeDtypeStruct((N, H), _f32),
                   jax.ShapeDtypeStruct((N, H), _f32)],
    )(x_pad, w_src, w_dst)


def _tc_node(a, b, cnt_d, cnt_s, wi, wo):
    def body(a_ref, b_ref, cd_ref, cs_ref, wi_ref, wo_ref, ta_ref, tb_ref):
        cd = jnp.maximum(cd_ref[:, 0:1], 1.0)
        cs = jnp.maximum(cs_ref[:, 0:1], 1.0)
        ta_ref[...] = _dot(a_ref[...] / cd, wi_ref[...])
        tb_ref[...] = _dot(b_ref[...] / cs, wo_ref[...])

    return pl.pallas_call(
        body,
        grid=(N // BN,),
        in_specs=[pl.BlockSpec((BN, H), lambda i: (i, 0)),
                  pl.BlockSpec((BN, H), lambda i: (i, 0)),
                  pl.BlockSpec((BN, WC), lambda i: (i, 0)),
                  pl.BlockSpec((BN, WC), lambda i: (i, 0)),
                  _w_spec(H, H), _w_spec(H, H)],
        out_specs=[pl.BlockSpec((BN, H), lambda i: (i, 0)),
                   pl.BlockSpec((BN, H), lambda i: (i, 0))],
        out_shape=[jax.ShapeDtypeStruct((N, H), _f32),
                   jax.ShapeDtypeStruct((N, H), _f32)],
    )(a, b, cnt_d, cnt_s, wi, wo)


def _tc_combine_enc(e_pad, ga, gb, w_e_pad, enc_b, ws1, bs1):
    def body(e_ref, ga_ref, gb_ref, we_ref, eb_ref, w_ref, b_ref,
             h_ref, s_ref):
        hv = _leaky(_dot(e_ref[...], we_ref[...]) + eb_ref[...]
                    + ga_ref[...] + gb_ref[...])
        h_ref[...] = hv
        s_ref[...] = _dot(hv, w_ref[...]) + b_ref[...]

    return pl.pallas_call(
        body,
        grid=(EG,),
        in_specs=[pl.BlockSpec((BE, 8), lambda i: (i, 0)),
                  pl.BlockSpec((BE, H), lambda i: (i, 0)),
                  pl.BlockSpec((BE, H), lambda i: (i, 0)),
                  _w_spec(8, H), _w_spec(1, H), _w_spec(H, H), _w_spec(1, H)],
        out_specs=[pl.BlockSpec((BE, H), lambda i: (i, 0)),
                   pl.BlockSpec((BE, H), lambda i: (i, 0))],
        out_shape=[jax.ShapeDtypeStruct((E_PAD, H), _f32),
                   jax.ShapeDtypeStruct((E_PAD, H), _f32)],
    )(e_pad, ga, gb, w_e_pad, enc_b, ws1, bs1)


def _tc_combine_conv(s, ga, gb, w_next, b_next):
    def body(s_ref, ga_ref, gb_ref, w_ref, b_ref, h_ref, so_ref):
        hv = _leaky(s_ref[...] + ga_ref[...] + gb_ref[...])
        h_ref[...] = hv
        so_ref[...] = _dot(hv, w_ref[...]) + b_ref[...]

    return pl.pallas_call(
        body,
        grid=(EG,),
        in_specs=[pl.BlockSpec((BE, H), lambda i: (i, 0)),
                  pl.BlockSpec((BE, H), lambda i: (i, 0)),
                  pl.BlockSpec((BE, H), lambda i: (i, 0)),
                  _w_spec(H, H), _w_spec(1, H)],
        out_specs=[pl.BlockSpec((BE, H), lambda i: (i, 0)),
                   pl.BlockSpec((BE, H), lambda i: (i, 0))],
        out_shape=[jax.ShapeDtypeStruct((E_PAD, H), _f32),
                   jax.ShapeDtypeStruct((E_PAD, H), _f32)],
    )(s, ga, gb, w_next, b_next)


def _tc_combine_mlp(s, ga, gb, m1w, m1b, m2w, m2b, m3w, m3b, m4w, m4b):
    def body(s_ref, ga_ref, gb_ref, w1_ref, b1_ref, w2_ref, b2_ref,
             w3_ref, b3_ref, w4_ref, b4_ref, o_ref):
        hv = _leaky(s_ref[...] + ga_ref[...] + gb_ref[...])
        hv = _leaky(_dot(hv, w1_ref[...]) + b1_ref[...])
        hv = _leaky(_dot(hv, w2_ref[...]) + b2_ref[...])
        hv = _leaky(_dot(hv, w3_ref[...]) + b3_ref[...])
        o_ref[...] = jnp.sum(hv * w4_ref[...], axis=1) + b4_ref[0, 0]

    return pl.pallas_call(
        body,
        grid=(EG,),
        in_specs=[pl.BlockSpec((BE, H), lambda i: (i, 0)),
                  pl.BlockSpec((BE, H), lambda i: (i, 0)),
                  pl.BlockSpec((BE, H), lambda i: (i, 0)),
                  _w_spec(H, H), _w_spec(1, H),
                  _w_spec(H, H), _w_spec(1, H),
                  _w_spec(H, 32), _w_spec(1, 32),
                  _w_spec(1, 32),
                  pl.BlockSpec((1, 1), lambda i: (0, 0),
                               memory_space=pltpu.SMEM)],
        out_specs=pl.BlockSpec((BE,), lambda i: (i,)),
        out_shape=jax.ShapeDtypeStruct((E,), _f32),
    )(s, ga, gb, m1w, m1b, m2w, m2b, m3w, m3b, m4w, m4b)


def kernel(x, edge_index, raw_edge_attr, enc_W, enc_b,
           c1_Ws, c1_bs, c1_Wi, c1_Wo,
           c2_Ws, c2_bs, c2_Wi, c2_Wo,
           c3_Ws, c3_bs, c3_Wi, c3_Wo,
           m1_W, m1_b, m2_W, m2_b, m3_W, m3_b, m4_W, m4_b):
    src = edge_index[0].astype(jnp.int32)
    dst = edge_index[1].astype(jnp.int32)
    npad = E_PAD - E
    pad_g = jnp.zeros((npad,), jnp.int32)
    pad_s = jnp.full((npad,), N, jnp.int32)
    src_g = jnp.concatenate([src, pad_g]).reshape(NBLK, GW)
    dst_g = jnp.concatenate([dst, pad_g]).reshape(NBLK, GW)
    src_s = jnp.concatenate([src, pad_s]).reshape(NBLK, GW)
    dst_s = jnp.concatenate([dst, pad_s]).reshape(NBLK, GW)

    zeros64 = jnp.zeros((GW, H), _f32)
    zeros8 = jnp.zeros((GW, WC), _f32)
    ones8 = jnp.ones((GW, WC), _f32)

    x_pad = jnp.pad(x, ((0, 0), (0, 3)))
    w_src = jnp.pad(enc_W[0:13], ((0, 3), (0, 0)))
    w_dst = jnp.pad(enc_W[13:26], ((0, 3), (0, 0)))
    e_pad = jnp.pad(raw_edge_attr, ((0, 0), (0, 7)))
    w_e_pad = jnp.pad(enc_W[26:27], ((0, 7), (0, 0)))

    cnt_d, cnt_s = _sc_count2(dst_s, src_s, ones8, zeros8)

    xa, xb = _tc_enc_node(x_pad, w_src, w_dst)
    ga, gb = _sc_gather2(xa, xb, src_g, dst_g)
    h, s = _tc_combine_enc(e_pad, ga, gb, w_e_pad, enc_b.reshape(1, H),
                           c1_Ws, c1_bs.reshape(1, H))

    convs = [(c1_Wi, c1_Wo, c2_Ws, c2_bs), (c2_Wi, c2_Wo, c3_Ws, c3_bs)]
    for wi, wo, ws_n, bs_n in convs:
        a = _sc_scatter_sum(h, dst_s, zeros64)
        b = _sc_scatter_sum(h, src_s, zeros64)
        ta, tb = _tc_node(a, b, cnt_d, cnt_s, wi, wo)
        ga, gb = _sc_gather2(ta, tb, src_g, dst_g)
        h, s = _tc_combine_conv(s, ga, gb, ws_n, bs_n.reshape(1, H))

    a = _sc_scatter_sum(h, dst_s, zeros64)
    b = _sc_scatter_sum(h, src_s, zeros64)
    ta, tb = _tc_node(a, b, cnt_d, cnt_s, c3_Wi, c3_Wo)
    ga, gb = _sc_gather2(ta, tb, src_g, dst_g)
    out = _tc_combine_mlp(s, ga, gb,
                          m1_W, m1_b.reshape(1, H),
                          m2_W, m2_b.reshape(1, H),
                          m3_W, m3_b.reshape(1, 32),
                          m4_W.reshape(1, 32), m4_b.reshape(1, 1))
    return out


# speedup vs baseline: 5.0041x; 1.0028x over previous
import functools

import jax
import jax.numpy as jnp
from jax import lax
from jax.experimental import pallas as pl
from jax.experimental.pallas import tpu as pltpu
from jax.experimental.pallas import tpu_sc as plsc

N = 50000
E = 800000
H = 64
WC = 8
NC = 2
NS = 16
LANES = 16
HN = N // NC
ACC_ROWS = HN + 8
GW = 128
E_PAD = 802816
NBLK = E_PAD // GW
NBT = NBLK // NS
GCH = 2
GBLK = GCH * GW
NGB = E_PAD // GBLK // (NC * NS)
CCH = 8
BE = 4096
EG = E_PAD // BE
BN = 2000

_f32 = jnp.float32


def _leaky(v):
    return jnp.maximum(v, 0.2 * v)


def _dot(a, b):
    return jnp.dot(a, b, preferred_element_type=_f32,
                   precision=jax.lax.Precision.HIGHEST)


def _sc_mesh():
    return plsc.VectorSubcoreMesh(core_axis_name="core",
                                  subcore_axis_name="subcore")


_SC_PARAMS = pltpu.CompilerParams(use_tc_tiling_on_sc=False)


def _transform_row(i_ref, row_in, l_ref, row_out, base):
    for t in range(GW // LANES):
        v = i_ref[pl.ds(row_in, 1), pl.ds(t * LANES, LANES)]
        local = v - base
        local = jnp.where((local < 0) | (local >= HN), HN, local)
        l_ref[pl.ds(row_out, 1), pl.ds(t * LANES, LANES)] = local


def _zero_acc(zeros_hbm, acc, sid, rows, zr):
    per_tile = rows // NS
    ncp = per_tile // zr + 1
    for c in range(ncp):
        start = jnp.minimum(sid * per_tile + c * zr, rows - zr)
        pltpu.sync_copy(zeros_hbm, acc.at[pl.ds(start, zr)])


def _acc_writeout(acc, out_hbm, cid, sid):
    for k in range(13):
        c = k * NS + sid

        @pl.when(c < 200)
        def _():
            pltpu.sync_copy(acc.at[pl.ds(c * 125, 125)],
                            out_hbm.at[pl.ds(cid * HN + c * 125, 125)])


def _sc_scatter_sum(vals, idx2, zeros64):
    @functools.partial(
        pl.kernel,
        out_type=jax.ShapeDtypeStruct((N, H), _f32),
        mesh=_sc_mesh(),
        compiler_params=_SC_PARAMS,
        scratch_types=[
            pltpu.VMEM_SHARED((ACC_ROWS, H), _f32),
            pltpu.VMEM((1, GW), jnp.int32),
        ],
    )
    def kern(vals_hbm, idx_hbm, z_hbm, out_hbm, acc, idxl):
        cid = lax.axis_index("core")
        sid = lax.axis_index("subcore")
        base = cid * HN
        _zero_acc(z_hbm, acc, sid, ACC_ROWS, GW)
        plsc.subcore_barrier()

        def body(v_vmem, i_vmem):
            _transform_row(i_vmem, 0, idxl, 0, base)
            pltpu.sync_copy(v_vmem, acc.at[idxl.at[0]], add=True)

        pltpu.emit_pipeline(
            body,
            grid=(NBLK,),
            in_specs=[
                pl.BlockSpec((GW, H), lambda i: (i, 0)),
                pl.BlockSpec((1, GW), lambda i: (i, 0)),
            ],
            out_specs=[],
            core_axis_name="subcore",
            dimension_semantics=(pltpu.PARALLEL,),
        )(vals_hbm, idx_hbm)
        plsc.subcore_barrier()
        _acc_writeout(acc, out_hbm, cid, sid)

    return kern(vals, idx2, zeros64)


def _sc_count2(idxd2, idxs2, ones8, zeros8):
    @functools.partial(
        pl.kernel,
        out_type=[jax.ShapeDtypeStruct((N, WC), _f32),
                  jax.ShapeDtypeStruct((N, WC), _f32)],
        mesh=_sc_mesh(),
        compiler_params=_SC_PARAMS,
        scratch_types=[
            pltpu.VMEM_SHARED((ACC_ROWS, WC), _f32),
            pltpu.VMEM_SHARED((ACC_ROWS, WC), _f32),
            pltpu.VMEM((GW, WC), _f32),
            pltpu.VMEM((CCH, GW), jnp.int32),
            pltpu.VMEM((CCH, GW), jnp.int32),
            pltpu.SemaphoreType.DMA,
        ],
    )
    def kern(id_hbm, is_hbm, ones_hbm, z_hbm, outd_hbm, outs_hbm,
             accd, accs, ones_v, lbd, lbs, sem):
        cid = lax.axis_index("core")
        sid = lax.axis_index("subcore")
        base = cid * HN
        pltpu.sync_copy(ones_hbm, ones_v)
        _zero_acc(z_hbm, accd, sid, ACC_ROWS, GW)
        _zero_acc(z_hbm, accs, sid, ACC_ROWS, GW)
        plsc.subcore_barrier()

        def body(id_vmem, is_vmem):
            for j in range(CCH):
                _transform_row(id_vmem, j, lbd, j, base)
                _transform_row(is_vmem, j, lbs, j, base)
            for j in range(CCH):
                pltpu.async_copy(ones_v, accd.at[lbd.at[j]], sem, add=True)
                pltpu.async_copy(ones_v, accs.at[lbs.at[j]], sem, add=True)
            for j in range(CCH):
                pltpu.make_async_copy(ones_v, accd.at[lbd.at[j]], sem).wait()
                pltpu.make_async_copy(ones_v, accs.at[lbs.at[j]], sem).wait()

        pltpu.emit_pipeline(
            body,
            grid=(NBLK // CCH,),
            in_specs=[pl.BlockSpec((CCH, GW), lambda i: (i, 0)),
                      pl.BlockSpec((CCH, GW), lambda i: (i, 0))],
            out_specs=[],
            core_axis_name="subcore",
            dimension_semantics=(pltpu.PARALLEL,),
        )(id_hbm, is_hbm)
        plsc.subcore_barrier()
        _acc_writeout(accd, outd_hbm, cid, sid)
        _acc_writeout(accs, outs_hbm, cid, sid)

    return kern(idxd2, idxs2, ones8, zeros8)


def _sc_gather2(ta, tb, src2, dst2):
    @functools.partial(
        pl.kernel,
        out_type=[jax.ShapeDtypeStruct((E_PAD, H), _f32),
                  jax.ShapeDtypeStruct((E_PAD, H), _f32)],
        mesh=_sc_mesh(),
        compiler_params=_SC_PARAMS,
        scratch_types=[pltpu.SemaphoreType.DMA],
    )
    def kern(ta_hbm, tb_hbm, si_hbm, di_hbm, ga_hbm, gb_hbm, sem):
        def body(si_vmem, di_vmem, ga_vmem, gb_vmem):
            for c in range(GCH):
                pltpu.async_copy(ta_hbm.at[si_vmem.at[c]],
                                 ga_vmem.at[pl.ds(c * GW, GW)], sem)
                pltpu.async_copy(tb_hbm.at[di_vmem.at[c]],
                                 gb_vmem.at[pl.ds(c * GW, GW)], sem)
            for c in range(GCH):
                pltpu.make_async_copy(ta_hbm.at[si_vmem.at[c]],
                                      ga_vmem.at[pl.ds(c * GW, GW)],
                                      sem).wait()
                pltpu.make_async_copy(tb_hbm.at[di_vmem.at[c]],
                                      gb_vmem.at[pl.ds(c * GW, GW)],
                                      sem).wait()

        pltpu.emit_pipeline(
            body,
            grid=(NBLK // GCH,),
            in_specs=[pl.BlockSpec((GCH, GW), lambda i: (i, 0)),
                      pl.BlockSpec((GCH, GW), lambda i: (i, 0))],
            out_specs=[pl.BlockSpec((GBLK, H), lambda i: (i, 0)),
                       pl.BlockSpec((GBLK, H), lambda i: (i, 0))],
            core_axis_name=("core", "subcore"),
            dimension_semantics=(pltpu.PARALLEL,),
        )(si_hbm, di_hbm, ga_hbm, gb_hbm)

    return kern(ta, tb, src2, dst2)


def _w_spec(r, c):
    return pl.BlockSpec((r, c), lambda i: (0, 0))


def _tc_enc_node(x_pad, w_src, w_dst):
    def body(x_ref, ws_ref, wd_ref, xa_ref, xb_ref):
        xv = x_ref[...]
        xa_ref[...] = _dot(xv, ws_ref[...])
        xb_ref[...] = _dot(xv, wd_ref[...])

    return pl.pallas_call(
        body,
        grid=(N // BN,),
        in_specs=[pl.BlockSpec((BN, 16), lambda i: (i, 0)),
                  _w_spec(16, H), _w_spec(16, H)],
        out_specs=[pl.BlockSpec((BN, H), lambda i: (i, 0)),
                   pl.BlockSpec((BN, H), lambda i: (i, 0))],
        out_shape=[jax.ShapeDtypeStruct((N, H), _f32),
                   jax.ShapeDtypeStruct((N, H), _f32)],
    )(x_pad, w_src, w_dst)


def _tc_node(a, b, cnt_d, cnt_s, wi, wo):
    def body(a_ref, b_ref, cd_ref, cs_ref, wi_ref, wo_ref, ta_ref, tb_ref):
        cd = jnp.maximum(cd_ref[:, 0:1], 1.0)
        cs = jnp.maximum(cs_ref[:, 0:1], 1.0)
        ta_ref[...] = _dot(a_ref[...] / cd, wi_ref[...])
        tb_ref[...] = _dot(b_ref[...] / cs, wo_ref[...])

    return pl.pallas_call(
        body,
        grid=(N // BN,),
        in_specs=[pl.BlockSpec((BN, H), lambda i: (i, 0)),
                  pl.BlockSpec((BN, H), lambda i: (i, 0)),
                  pl.BlockSpec((BN, WC), lambda i: (i, 0)),
                  pl.BlockSpec((BN, WC), lambda i: (i, 0)),
                  _w_spec(H, H), _w_spec(H, H)],
        out_specs=[pl.BlockSpec((BN, H), lambda i: (i, 0)),
                   pl.BlockSpec((BN, H), lambda i: (i, 0))],
        out_shape=[jax.ShapeDtypeStruct((N, H), _f32),
                   jax.ShapeDtypeStruct((N, H), _f32)],
    )(a, b, cnt_d, cnt_s, wi, wo)


def _tc_combine_enc(e_pad, ga, gb, w_e_pad, enc_b):
    def body(e_ref, ga_ref, gb_ref, we_ref, eb_ref, h_ref):
        h_ref[...] = _leaky(_dot(e_ref[...], we_ref[...]) + eb_ref[...]
                            + ga_ref[...] + gb_ref[...])

    return pl.pallas_call(
        body,
        grid=(EG,),
        in_specs=[pl.BlockSpec((BE, 8), lambda i: (i, 0)),
                  pl.BlockSpec((BE, H), lambda i: (i, 0)),
                  pl.BlockSpec((BE, H), lambda i: (i, 0)),
                  _w_spec(8, H), _w_spec(1, H)],
        out_specs=pl.BlockSpec((BE, H), lambda i: (i, 0)),
        out_shape=jax.ShapeDtypeStruct((E_PAD, H), _f32),
    )(e_pad, ga, gb, w_e_pad, enc_b)


def _tc_combine_conv(h_prev, ga, gb, ws, bs):
    def body(hp_ref, ga_ref, gb_ref, w_ref, b_ref, h_ref):
        h_ref[...] = _leaky(_dot(hp_ref[...], w_ref[...]) + b_ref[...]
                            + ga_ref[...] + gb_ref[...])

    return pl.pallas_call(
        body,
        grid=(EG,),
        in_specs=[pl.BlockSpec((BE, H), lambda i: (i, 0)),
                  pl.BlockSpec((BE, H), lambda i: (i, 0)),
                  pl.BlockSpec((BE, H), lambda i: (i, 0)),
                  _w_spec(H, H), _w_spec(1, H)],
        out_specs=pl.BlockSpec((BE, H), lambda i: (i, 0)),
        out_shape=jax.ShapeDtypeStruct((E_PAD, H), _f32),
    )(h_prev, ga, gb, ws, bs)


def _tc_combine_mlp(h_prev, ga, gb, ws, bs,
                    m1w, m1b, m2w, m2b, m3w, m3b, m4w, m4b):
    def body(hp_ref, ga_ref, gb_ref, w_ref, b_ref, w1_ref, b1_ref,
             w2_ref, b2_ref, w3_ref, b3_ref, w4_ref, b4_ref, o_ref):
        hv = _leaky(_dot(hp_ref[...], w_ref[...]) + b_ref[...]
                    + ga_ref[...] + gb_ref[...])
        hv = _leaky(_dot(hv, w1_ref[...]) + b1_ref[...])
        hv = _leaky(_dot(hv, w2_ref[...]) + b2_ref[...])
        hv = _leaky(_dot(hv, w3_ref[...]) + b3_ref[...])
        o_ref[...] = jnp.sum(hv * w4_ref[...], axis=1) + b4_ref[0, 0]

    return pl.pallas_call(
        body,
        grid=(EG,),
        in_specs=[pl.BlockSpec((BE, H), lambda i: (i, 0)),
                  pl.BlockSpec((BE, H), lambda i: (i, 0)),
                  pl.BlockSpec((BE, H), lambda i: (i, 0)),
                  _w_spec(H, H), _w_spec(1, H),
                  _w_spec(H, H), _w_spec(1, H),
                  _w_spec(H, H), _w_spec(1, H),
                  _w_spec(H, 32), _w_spec(1, 32),
                  _w_spec(1, 32),
                  pl.BlockSpec((1, 1), lambda i: (0, 0),
                               memory_space=pltpu.SMEM)],
        out_specs=pl.BlockSpec((BE,), lambda i: (i,)),
        out_shape=jax.ShapeDtypeStruct((E,), _f32),
    )(h_prev, ga, gb, ws, bs, m1w, m1b, m2w, m2b, m3w, m3b, m4w, m4b)


def kernel(x, edge_index, raw_edge_attr, enc_W, enc_b,
           c1_Ws, c1_bs, c1_Wi, c1_Wo,
           c2_Ws, c2_bs, c2_Wi, c2_Wo,
           c3_Ws, c3_bs, c3_Wi, c3_Wo,
           m1_W, m1_b, m2_W, m2_b, m3_W, m3_b, m4_W, m4_b):
    src = edge_index[0].astype(jnp.int32)
    dst = edge_index[1].astype(jnp.int32)
    npad = E_PAD - E
    pad_g = jnp.zeros((npad,), jnp.int32)
    pad_s = jnp.full((npad,), N, jnp.int32)
    src_g = jnp.concatenate([src, pad_g]).reshape(NBLK, GW)
    dst_g = jnp.concatenate([dst, pad_g]).reshape(NBLK, GW)
    src_s = jnp.concatenate([src, pad_s]).reshape(NBLK, GW)
    dst_s = jnp.concatenate([dst, pad_s]).reshape(NBLK, GW)

    zeros64 = jnp.zeros((GW, H), _f32)
    zeros8 = jnp.zeros((GW, WC), _f32)
    ones8 = jnp.ones((GW, WC), _f32)

    x_pad = jnp.pad(x, ((0, 0), (0, 3)))
    w_src = jnp.pad(enc_W[0:13], ((0, 3), (0, 0)))
    w_dst = jnp.pad(enc_W[13:26], ((0, 3), (0, 0)))
    e_pad = jnp.pad(raw_edge_attr, ((0, 0), (0, 7)))
    w_e_pad = jnp.pad(enc_W[26:27], ((0, 7), (0, 0)))

    cnt_d, cnt_s = _sc_count2(dst_s, src_s, ones8, zeros8)

    xa, xb = _tc_enc_node(x_pad, w_src, w_dst)
    ga, gb = _sc_gather2(xa, xb, src_g, dst_g)
    h = _tc_combine_enc(e_pad, ga, gb, w_e_pad, enc_b.reshape(1, H))

    convs = [(c1_Ws, c1_bs, c1_Wi, c1_Wo), (c2_Ws, c2_bs, c2_Wi, c2_Wo)]
    for ws, bs, wi, wo in convs:
        a = _sc_scatter_sum(h, dst_s, zeros64)
        b = _sc_scatter_sum(h, src_s, zeros64)
        ta, tb = _tc_node(a, b, cnt_d, cnt_s, wi, wo)
        ga, gb = _sc_gather2(ta, tb, src_g, dst_g)
        h = _tc_combine_conv(h, ga, gb, ws, bs.reshape(1, H))

    a = _sc_scatter_sum(h, dst_s, zeros64)
    b = _sc_scatter_sum(h, src_s, zeros64)
    ta, tb = _tc_node(a, b, cnt_d, cnt_s, c3_Wi, c3_Wo)
    ga, gb = _sc_gather2(ta, tb, src_g, dst_g)
    out = _tc_combine_mlp(h, ga, gb, c3_Ws, c3_bs.reshape(1, H),
                          m1_W, m1_b.reshape(1, H),
                          m2_W, m2_b.reshape(1, H),
                          m3_W, m3_b.reshape(1, 32),
                          m4_W.reshape(1, 32), m4_b.reshape(1, 1))
    return out
